# baseline (speedup 1.0000x reference)
import functools
import math

import jax
import jax.numpy as jnp
from jax import lax
from jax.experimental import pallas as pl
from jax.experimental.pallas import tpu as pltpu
from jax.experimental.pallas import tpu_sc as plsc

D_MODEL = 64
SCALE = math.sqrt(D_MODEL)


@functools.lru_cache(maxsize=None)
def _make_lookup(V: int, B: int):
    info = plsc.get_sparse_core_info()
    NC, NS, L = info.num_cores, info.num_subcores, info.num_lanes
    NW = NC * NS
    assert B % NW == 0
    b_per_w = B // NW
    C = 320
    assert b_per_w % (2 * C) == 0 and C % 8 == 0
    n_chunks = b_per_w // C
    n_steps = n_chunks // 2

    mesh = plsc.VectorSubcoreMesh(core_axis_name="c", subcore_axis_name="s")

    @functools.partial(
        pl.kernel,
        mesh=mesh,
        out_type=jax.ShapeDtypeStruct((B, D_MODEL), jnp.float32),
        compiler_params=pltpu.CompilerParams(use_tc_tiling_on_sc=False),
        scratch_types=[
            pltpu.VMEM((b_per_w,), jnp.int32),
            pltpu.VMEM((C, D_MODEL), jnp.float32),
            pltpu.VMEM((C, D_MODEL), jnp.float32),
            pltpu.VMEM((C, D_MODEL), jnp.float32),
            pltpu.VMEM((C, D_MODEL), jnp.float32),
            pltpu.SemaphoreType.DMA,
            pltpu.SemaphoreType.DMA,
            pltpu.SemaphoreType.DMA,
            pltpu.SemaphoreType.DMA,
        ],
    )
    def lookup(idx_hbm, table_hbm, out_hbm, idx_all, ga, gb, wa, wb,
               gsa, gsb, wsa, wsb):
        wid = lax.axis_index("s") * NC + lax.axis_index("c")
        base = wid * b_per_w
        pltpu.sync_copy(idx_hbm.at[pl.ds(base, b_per_w)], idx_all)

        def gather_desc(g, buf, sem):
            src = table_hbm.at[idx_all.at[pl.ds(g * C, C)]]
            return pltpu.make_async_copy(src, buf, sem)

        def write_desc(g, buf, sem):
            return pltpu.make_async_copy(
                buf, out_hbm.at[pl.ds(base + g * C, C)], sem)

        def scale(src, dst):
            def body(r, c2):
                for kk in range(D_MODEL // L):
                    sl = pl.ds(kk * L, L)
                    dst[r, sl] = src[r, sl] * SCALE
                return c2
            lax.fori_loop(0, C, body, 0, unroll=4)

        gather_desc(0, ga, gsa).start()
        gather_desc(1, gb, gsb).start()

        def step(t, carry):
            for g, gbuf, gsem, wbuf, wsem in (
                    (2 * t, ga, gsa, wa, wsa),
                    (2 * t + 1, gb, gsb, wb, wsb)):
                gather_desc(g, gbuf, gsem).wait()

                @pl.when(t > 0)
                def _():
                    write_desc(g - 2, wbuf, wsem).wait()

                scale(gbuf, wbuf)

                @pl.when(t < n_steps - 1)
                def _():
                    gather_desc(g + 2, gbuf, gsem).start()

                write_desc(g, wbuf, wsem).start()
            return carry

        lax.fori_loop(0, n_steps, step, 0)
        write_desc(n_chunks - 2, wa, wsa).wait()
        write_desc(n_chunks - 1, wb, wsb).wait()

    return lookup


def kernel(x, table):
    B = x.size
    idx = x.reshape(B).astype(jnp.int32)
    out = _make_lookup(table.shape[0], B)(idx, table)
    return out.reshape(x.shape + (D_MODEL,))
